# Initial kernel scaffold; baseline (speedup 1.0000x reference)
import jax
import jax.numpy as jnp
from jax.experimental import pallas as pl


def kernel(x):
    raise NotImplementedError("write your pallas kernel here")


# speedup vs baseline: 3.9955x; 3.9955x over previous
import jax
import jax.numpy as jnp
from jax import lax
from jax.experimental import pallas as pl
from jax.experimental.pallas import tpu as pltpu
from jax.experimental.pallas import tpu_sc as plsc

N = 4096
D = 128
BLK = 256
CUTOFF_DIST = 1.4
BIG = 1e30


def _tc_body(x_ref, xn_ref, idx_ref):
    i = pl.program_id(0)
    x = x_ref[...]
    nrm = jnp.sqrt(jnp.sum(x * x, axis=1, keepdims=True))
    xn = x / jnp.maximum(nrm, 1e-12)
    xr = x_ref[pl.ds(i * BLK, BLK), :]
    rnrm = jnp.sqrt(jnp.sum(xr * xr, axis=1, keepdims=True))
    rows = xr / jnp.maximum(rnrm, 1e-12)
    xn_ref[...] = rows

    sim = lax.dot_general(
        rows, xn, (((1,), (1,)), ((), ())), preferred_element_type=jnp.float32
    )
    sim = jnp.minimum(sim, 1.0)
    dist = jnp.sqrt(2.0 - 2.0 * sim)
    dist = jnp.maximum(dist, 1e-08)

    col = lax.broadcasted_iota(jnp.int32, (BLK, N), 1)
    row = i * BLK + lax.broadcasted_iota(jnp.int32, (BLK, N), 0)
    valid = (dist < CUTOFF_DIST) & (col != row)
    score = jnp.where(valid, dist, BIG)
    m = jnp.min(score, axis=1, keepdims=True)
    idx = jnp.min(jnp.where(score == m, col, N), axis=1)
    idx_ref[0, 0, :] = idx.astype(jnp.int32)


def _tc_call(x):
    grid = N // BLK
    return pl.pallas_call(
        _tc_body,
        grid=(grid,),
        in_specs=[pl.BlockSpec((N, D), lambda i: (0, 0))],
        out_specs=[
            pl.BlockSpec((BLK, D), lambda i: (i, 0)),
            pl.BlockSpec((1, 1, BLK), lambda i: (i, 0, 0)),
        ],
        out_shape=[
            jax.ShapeDtypeStruct((N, D), jnp.float32),
            jax.ShapeDtypeStruct((grid, 1, BLK), jnp.int32),
        ],
    )(x)


ROWS_PER_W = N // 32
GROUP = 16
SLAB = GROUP * N


def _sc_body(idx_hbm, out_hbm, idx_v, slab_v):
    nc = 2
    wid = lax.axis_index("s") * nc + lax.axis_index("c")
    base = wid * ROWS_PER_W
    pltpu.sync_copy(idx_hbm.at[pl.ds(base, ROWS_PER_W)], idx_v)

    ones = jnp.ones((16,), jnp.float32)
    zeros = jnp.zeros((16,), jnp.float32)
    lanes = lax.iota(jnp.int32, 16)

    def fill(j, carry):
        slab_v[pl.ds(j * 16, 16)] = ones
        return carry

    lax.fori_loop(0, SLAB // 16, fill, 0)

    for g in range(ROWS_PER_W // GROUP):
        cols = idx_v[pl.ds(g * GROUP, GROUP)]
        flat = lanes * N + cols
        plsc.store_scatter(slab_v, [flat], zeros)
        pltpu.sync_copy(
            slab_v, out_hbm.at[pl.ds((base + g * GROUP) * N, SLAB)]
        )
        plsc.store_scatter(slab_v, [flat], ones)


def _sc_call(neg_idx):
    mesh = plsc.VectorSubcoreMesh(core_axis_name="c", subcore_axis_name="s")
    return pl.kernel(
        _sc_body,
        out_type=jax.ShapeDtypeStruct((N * N,), jnp.float32),
        mesh=mesh,
        scratch_types=[
            pltpu.VMEM((ROWS_PER_W,), jnp.int32),
            pltpu.VMEM((SLAB,), jnp.float32),
        ],
        compiler_params=pltpu.CompilerParams(needs_layout_passes=False),
    )(neg_idx)


@jax.jit
def kernel(x):
    xn, idx_blocks = _tc_call(x)
    neg_idx = idx_blocks.reshape((N,))
    clm = _sc_call(neg_idx).reshape((N, N))
    return (xn, clm)


# speedup vs baseline: 6.1581x; 1.5413x over previous
import jax
import jax.numpy as jnp
from jax import lax
from jax.experimental import pallas as pl
from jax.experimental.pallas import tpu as pltpu
from jax.experimental.pallas import tpu_sc as plsc

N = 4096
D = 128
BLK = 256
CUTOFF_DIST = 1.4
BIG = 1e30


def _tc_body(x_ref, xn_ref, idx_ref):
    i = pl.program_id(0)
    x = x_ref[...]
    nrm = jnp.sqrt(jnp.sum(x * x, axis=1, keepdims=True))
    xn = x / jnp.maximum(nrm, 1e-12)
    xr = x_ref[pl.ds(i * BLK, BLK), :]
    rnrm = jnp.sqrt(jnp.sum(xr * xr, axis=1, keepdims=True))
    rows = xr / jnp.maximum(rnrm, 1e-12)
    xn_ref[...] = rows

    sim = lax.dot_general(
        rows, xn, (((1,), (1,)), ((), ())), preferred_element_type=jnp.float32
    )
    sim = jnp.minimum(sim, 1.0)
    dist = jnp.sqrt(2.0 - 2.0 * sim)
    dist = jnp.maximum(dist, 1e-08)

    col = lax.broadcasted_iota(jnp.int32, (BLK, N), 1)
    row = i * BLK + lax.broadcasted_iota(jnp.int32, (BLK, N), 0)
    valid = (dist < CUTOFF_DIST) & (col != row)
    score = jnp.where(valid, dist, BIG)
    m = jnp.min(score, axis=1, keepdims=True)
    idx = jnp.min(jnp.where(score == m, col, N), axis=1)
    idx_ref[0, 0, :] = idx.astype(jnp.int32)


def _tc_call(x):
    grid = N // BLK
    return pl.pallas_call(
        _tc_body,
        grid=(grid,),
        in_specs=[pl.BlockSpec((N, D), lambda i: (0, 0))],
        out_specs=[
            pl.BlockSpec((BLK, D), lambda i: (i, 0)),
            pl.BlockSpec((1, 1, BLK), lambda i: (i, 0, 0)),
        ],
        out_shape=[
            jax.ShapeDtypeStruct((N, D), jnp.float32),
            jax.ShapeDtypeStruct((grid, 1, BLK), jnp.int32),
        ],
    )(x)


ROWS_PER_W = N // 32
GROUP = 16
SLAB = GROUP * N


def _sc_body(idx_hbm, out_hbm, idx_v, slab_v):
    nc = 2
    wid = lax.axis_index("s") * nc + lax.axis_index("c")
    base = wid * ROWS_PER_W
    pltpu.sync_copy(idx_hbm.at[pl.ds(base, ROWS_PER_W)], idx_v)

    ones = jnp.ones((16,), jnp.float32)
    zeros = jnp.zeros((16,), jnp.float32)
    lanes = lax.iota(jnp.int32, 16)

    def fill(j, carry):
        slab_v[j // (N // 16), pl.ds((j % (N // 16)) * 16, 16)] = ones
        return carry

    lax.fori_loop(0, SLAB // 16, fill, 0)

    for g in range(ROWS_PER_W // GROUP):
        cols = idx_v[pl.ds(g * GROUP, GROUP)]
        plsc.store_scatter(slab_v, [lanes, cols], zeros)
        pltpu.sync_copy(
            slab_v, out_hbm.at[pl.ds(base + g * GROUP, GROUP), :]
        )
        plsc.store_scatter(slab_v, [lanes, cols], ones)


def _sc_call(neg_idx):
    mesh = plsc.VectorSubcoreMesh(core_axis_name="c", subcore_axis_name="s")
    return pl.kernel(
        _sc_body,
        out_type=jax.ShapeDtypeStruct((N, N), jnp.float32),
        mesh=mesh,
        scratch_types=[
            pltpu.VMEM((ROWS_PER_W,), jnp.int32),
            pltpu.VMEM((GROUP, N), jnp.float32),
        ],
        compiler_params=pltpu.CompilerParams(needs_layout_passes=False),
    )(neg_idx)


@jax.jit
def kernel(x):
    xn, idx_blocks = _tc_call(x)
    neg_idx = idx_blocks.reshape((N,))
    clm = _sc_call(neg_idx)
    return (xn, clm)


# speedup vs baseline: 9.0826x; 1.4749x over previous
import jax
import jax.numpy as jnp
from jax import lax
from jax.experimental import pallas as pl
from jax.experimental.pallas import tpu as pltpu
from jax.experimental.pallas import tpu_sc as plsc

N = 4096
D = 128
BLK = 256
CUTOFF_DIST = 1.4
BIG = 1e30


SIM_CUT = 0.02000012993812561


def _tc_body(x_ref, xn_ref, idx_ref, xns_ref):
    i = pl.program_id(0)

    @pl.when(i == 0)
    def _():
        x = x_ref[...]
        nrm = jnp.sqrt(jnp.sum(x * x, axis=1, keepdims=True))
        xns_ref[...] = x / jnp.maximum(nrm, 1e-12)

    xn = xns_ref[...]
    rows = xns_ref[pl.ds(i * BLK, BLK), :]
    xn_ref[...] = rows

    sim = lax.dot_general(
        rows, xn, (((1,), (1,)), ((), ())), preferred_element_type=jnp.float32
    )
    col = lax.broadcasted_iota(jnp.int32, (BLK, N), 1)
    row = i * BLK + lax.broadcasted_iota(jnp.int32, (BLK, N), 0)
    valid = (sim > SIM_CUT) & (col != row)
    score = jnp.where(valid, sim, -2.0)
    m = jnp.max(score, axis=1, keepdims=True)
    idx = jnp.min(jnp.where(score == m, col, N), axis=1)
    idx_ref[0, 0, :] = idx.astype(jnp.int32)


def _tc_call(x):
    grid = N // BLK
    return pl.pallas_call(
        _tc_body,
        grid=(grid,),
        in_specs=[pl.BlockSpec((N, D), lambda i: (0, 0))],
        out_specs=[
            pl.BlockSpec((BLK, D), lambda i: (i, 0)),
            pl.BlockSpec((1, 1, BLK), lambda i: (i, 0, 0)),
        ],
        out_shape=[
            jax.ShapeDtypeStruct((N, D), jnp.float32),
            jax.ShapeDtypeStruct((grid, 1, BLK), jnp.int32),
        ],
        scratch_shapes=[pltpu.VMEM((N, D), jnp.float32)],
    )(x)


ROWS_PER_W = N // 32
GROUP = 16
SLAB = GROUP * N


def _sc_body(idx_hbm, out_hbm, idx_v, slab_v):
    nc = 2
    wid = lax.axis_index("s") * nc + lax.axis_index("c")
    base = wid * ROWS_PER_W
    pltpu.sync_copy(idx_hbm.at[pl.ds(base, ROWS_PER_W)], idx_v)

    ones = jnp.ones((16,), jnp.float32)
    zeros = jnp.zeros((16,), jnp.float32)
    lanes = lax.iota(jnp.int32, 16)

    def fill(j, carry):
        slab_v[j // (N // 16), pl.ds((j % (N // 16)) * 16, 16)] = ones
        return carry

    lax.fori_loop(0, SLAB // 16, fill, 0)

    for g in range(ROWS_PER_W // GROUP):
        cols = idx_v[pl.ds(g * GROUP, GROUP)]
        plsc.store_scatter(slab_v, [lanes, cols], zeros)
        pltpu.sync_copy(
            slab_v, out_hbm.at[pl.ds(base + g * GROUP, GROUP), :]
        )
        plsc.store_scatter(slab_v, [lanes, cols], ones)


def _sc_call(neg_idx):
    mesh = plsc.VectorSubcoreMesh(core_axis_name="c", subcore_axis_name="s")
    return pl.kernel(
        _sc_body,
        out_type=jax.ShapeDtypeStruct((N, N), jnp.float32),
        mesh=mesh,
        scratch_types=[
            pltpu.VMEM((ROWS_PER_W,), jnp.int32),
            pltpu.VMEM((GROUP, N), jnp.float32),
        ],
        compiler_params=pltpu.CompilerParams(needs_layout_passes=False),
    )(neg_idx)


@jax.jit
def kernel(x):
    xn, idx_blocks = _tc_call(x)
    neg_idx = idx_blocks.reshape((N,))
    clm = _sc_call(neg_idx)
    return (xn, clm)


# speedup vs baseline: 10.9056x; 1.2007x over previous
import jax
import jax.numpy as jnp
from jax import lax
from jax.experimental import pallas as pl
from jax.experimental.pallas import tpu as pltpu
from jax.experimental.pallas import tpu_sc as plsc

N = 4096
D = 128
BLK = 256
CUTOFF_DIST = 1.4
BIG = 1e30


SIM_CUT = 0.02000012993812561


def _tc_body(x_ref, xn_ref, idx_ref, xns_ref):
    i = pl.program_id(0)

    @pl.when(i == 0)
    def _():
        x = x_ref[...]
        nrm = jnp.sqrt(jnp.sum(x * x, axis=1, keepdims=True))
        xns_ref[...] = x / jnp.maximum(nrm, 1e-12)

    xn = xns_ref[...]
    rows = xns_ref[pl.ds(i * BLK, BLK), :]
    xn_ref[...] = rows

    sim = lax.dot_general(
        rows, xn, (((1,), (1,)), ((), ())), preferred_element_type=jnp.float32
    )
    col = lax.broadcasted_iota(jnp.int32, (BLK, N), 1)
    row = i * BLK + lax.broadcasted_iota(jnp.int32, (BLK, N), 0)
    score = jnp.where(col == row, -2.0, sim)
    m = jnp.max(score, axis=1)
    am = jnp.argmax(score, axis=1).astype(jnp.int32)
    idx_ref[0, 0, :] = jnp.where(m > SIM_CUT, am, 0)


def _tc_call(x):
    grid = N // BLK
    return pl.pallas_call(
        _tc_body,
        grid=(grid,),
        in_specs=[pl.BlockSpec((N, D), lambda i: (0, 0))],
        out_specs=[
            pl.BlockSpec((BLK, D), lambda i: (i, 0)),
            pl.BlockSpec((1, 1, BLK), lambda i: (i, 0, 0)),
        ],
        out_shape=[
            jax.ShapeDtypeStruct((N, D), jnp.float32),
            jax.ShapeDtypeStruct((grid, 1, BLK), jnp.int32),
        ],
        scratch_shapes=[pltpu.VMEM((N, D), jnp.float32)],
    )(x)


ROWS_PER_W = N // 32
GROUP = 16
SLAB = GROUP * N


def _sc_body(idx_hbm, ones_hbm, out_hbm, idx_v, slab_v):
    nc = 2
    wid = lax.axis_index("s") * nc + lax.axis_index("c")
    base = wid * ROWS_PER_W
    pltpu.sync_copy(idx_hbm.at[pl.ds(base, ROWS_PER_W)], idx_v)
    pltpu.sync_copy(ones_hbm, slab_v)

    ones = jnp.ones((16,), jnp.float32)
    zeros = jnp.zeros((16,), jnp.float32)
    lanes = lax.iota(jnp.int32, 16)

    for g in range(ROWS_PER_W // GROUP):
        cols = idx_v[pl.ds(g * GROUP, GROUP)]
        plsc.store_scatter(slab_v, [lanes, cols], zeros)
        pltpu.sync_copy(
            slab_v, out_hbm.at[pl.ds(base + g * GROUP, GROUP), :]
        )
        plsc.store_scatter(slab_v, [lanes, cols], ones)


def _sc_call(neg_idx):
    mesh = plsc.VectorSubcoreMesh(core_axis_name="c", subcore_axis_name="s")
    ones2d = jnp.ones((GROUP, N), jnp.float32)
    return pl.kernel(
        _sc_body,
        out_type=jax.ShapeDtypeStruct((N, N), jnp.float32),
        mesh=mesh,
        scratch_types=[
            pltpu.VMEM((ROWS_PER_W,), jnp.int32),
            pltpu.VMEM((GROUP, N), jnp.float32),
        ],
        compiler_params=pltpu.CompilerParams(needs_layout_passes=False),
    )(neg_idx, ones2d)


@jax.jit
def kernel(x):
    xn, idx_blocks = _tc_call(x)
    neg_idx = idx_blocks.reshape((N,))
    clm = _sc_call(neg_idx)
    return (xn, clm)
